# Initial kernel scaffold; baseline (speedup 1.0000x reference)
import functools

import jax
import jax.numpy as jnp
from jax import lax
from jax.experimental import pallas as pl
from jax.experimental.pallas import tpu as pltpu
from jax.experimental.pallas import tpu_sc as plsc

_EMBED = 32
_NC = 2
_NS = 16
_NW = _NC * _NS
_ROW = 128
_B = 16384 * 50
_NROWS = _B // _ROW
_ROWS_PER_W = _NROWS // _NW
_NBUF = 4
_MAIN = _ROWS_PER_W - _NBUF


def _sc_body(table_hbm, idx_hbm, out_hbm, idx_v, rows, gsems, osems):
    wid = lax.axis_index("s") * _NC + lax.axis_index("c")
    rbase = wid * _ROWS_PER_W

    pltpu.sync_copy(idx_hbm.at[pl.ds(rbase, _ROWS_PER_W)], idx_v)

    def start_gather(j, b):
        pltpu.async_copy(table_hbm.at[idx_v.at[j]], rows[b], gsems[b])

    def wait_gather(j, b):
        pltpu.make_async_copy(table_hbm.at[idx_v.at[j]], rows[b], gsems[b]).wait()

    def out_copy(j, b):
        return pltpu.make_async_copy(
            rows[b], out_hbm.at[pl.ds((rbase + j) * _ROW, _ROW)], osems[b]
        )

    for b in range(_NBUF):
        start_gather(b, b)

    @pl.loop(0, _MAIN, step=_NBUF)
    def _(jj):
        for b in range(_NBUF):
            j = jj + b
            wait_gather(j, b)
            cp = out_copy(j, b)
            cp.start()
            cp.wait()
            start_gather(j + _NBUF, b)

    for b in range(_NBUF):
        j = _MAIN + b
        wait_gather(j, b)
        cp = out_copy(j, b)
        cp.start()
        cp.wait()


_sc_gather = functools.partial(
    pl.kernel,
    out_type=jax.ShapeDtypeStruct((_B, _EMBED), jnp.float32),
    mesh=plsc.VectorSubcoreMesh(core_axis_name="c", subcore_axis_name="s"),
    scratch_types=[
        pltpu.VMEM((_ROWS_PER_W, _ROW), jnp.int32),
        [pltpu.VMEM((_ROW, _EMBED), jnp.float32) for _ in range(_NBUF)],
        [pltpu.SemaphoreType.DMA for _ in range(_NBUF)],
        [pltpu.SemaphoreType.DMA for _ in range(_NBUF)],
    ],
)(_sc_body)


@jax.jit
def kernel(tokenid, table):
    idx = tokenid.reshape(_NROWS, _ROW)
    out = _sc_gather(table, idx)
    return out.reshape(tokenid.shape[0], tokenid.shape[1], _EMBED)


# speedup vs baseline: 1.1094x; 1.1094x over previous
import functools

import jax
import jax.numpy as jnp
from jax import lax
from jax.experimental import pallas as pl
from jax.experimental.pallas import tpu as pltpu
from jax.experimental.pallas import tpu_sc as plsc

_EMBED = 32
_NC = 2
_NS = 16
_NW = _NC * _NS
_ROW = 128
_B = 16384 * 50
_NROWS = _B // _ROW
_ROWS_PER_W = _NROWS // _NW
_NBUF = 4
_MAIN = _ROWS_PER_W - _NBUF


def _sc_body(table_hbm, idx_hbm, out_hbm, idx_v, rows, gsems, osems):
    wid = lax.axis_index("s") * _NC + lax.axis_index("c")
    rbase = wid * _ROWS_PER_W

    pltpu.sync_copy(idx_hbm.at[pl.ds(rbase, _ROWS_PER_W)], idx_v)

    def start_gather(j, b):
        pltpu.async_copy(table_hbm.at[idx_v.at[j]], rows[b], gsems[b])

    def wait_gather(j, b):
        pltpu.make_async_copy(table_hbm.at[idx_v.at[j]], rows[b], gsems[b]).wait()

    def out_copy(j, b):
        return pltpu.make_async_copy(
            rows[b], out_hbm.at[pl.ds((rbase + j) * _ROW, _ROW)], osems[b]
        )

    for b in range(_NBUF):
        start_gather(b, b)

    @pl.loop(0, _MAIN, step=_NBUF)
    def _(jj):
        for b in range(_NBUF):
            j = jj + b
            wait_gather(j, b)
            cp = out_copy(j, b)
            cp.start()
            cp.wait()
            start_gather(j + _NBUF, b)

    for b in range(_NBUF):
        j = _MAIN + b
        wait_gather(j, b)
        cp = out_copy(j, b)
        cp.start()
        cp.wait()


_sc_gather = functools.partial(
    pl.kernel,
    out_type=jax.ShapeDtypeStruct((_B, _EMBED), jnp.float32),
    mesh=plsc.VectorSubcoreMesh(core_axis_name="c", subcore_axis_name="s"),
    compiler_params=pltpu.CompilerParams(use_tc_tiling_on_sc=False),
    scratch_types=[
        pltpu.VMEM((_ROWS_PER_W, _ROW), jnp.int32),
        [pltpu.VMEM((_ROW, _EMBED), jnp.float32) for _ in range(_NBUF)],
        [pltpu.SemaphoreType.DMA for _ in range(_NBUF)],
        [pltpu.SemaphoreType.DMA for _ in range(_NBUF)],
    ],
)(_sc_body)


@jax.jit
def kernel(tokenid, table):
    idx = tokenid.reshape(_NROWS, _ROW)
    out = _sc_gather(table, idx)
    return out.reshape(tokenid.shape[0], tokenid.shape[1], _EMBED)


# speedup vs baseline: 1.6439x; 1.4818x over previous
import functools

import jax
import jax.numpy as jnp
from jax import lax
from jax.experimental import pallas as pl
from jax.experimental.pallas import tpu as pltpu
from jax.experimental.pallas import tpu_sc as plsc

_EMBED = 32
_SEQ = 50
_NTOK = 16384
_TB = 128
_NTB = _NTOK // _TB
_NBLK = _SEQ * _NTB
_NC = 2
_NS = 16
_NW = _NC * _NS
_BPW = _NBLK // _NW
_NBUF = 4
_NIB = 2 * _NBUF
_MAIN = _BPW - _NIB


def _sc_body(table_hbm, idx_hbm, out_hbm, idxv, rowsv, outv, isems, gsems, osems):
    wid = lax.axis_index("s") * _NC + lax.axis_index("c")
    base = wid * _BPW

    def blk(j):
        b = base + j
        return b >> 7, b & 127

    def idx_copy(j, i):
        s, t = blk(j)
        return pltpu.make_async_copy(
            idx_hbm.at[s, pl.ds(t * _TB, _TB)], idxv[i], isems[i]
        )

    def gather(j, i, b):
        return pltpu.make_async_copy(table_hbm.at[idxv[i]], rowsv[b], gsems[b])

    def out_copies(j, b):
        s, t = blk(j)
        return [
            pltpu.make_async_copy(
                outv[b].at[pl.ds(e8 * 8, 8)],
                out_hbm.at[s * (4 * _TB) + e8 * _TB + t],
                osems[b],
            )
            for e8 in range(4)
        ]

    tidx = [lax.iota(jnp.int32, 16) + (k * 16) for k in range(_TB // 16)]

    def transpose(b):
        @pl.loop(0, _EMBED)
        def _(e):
            eidx = jnp.full((16,), e, jnp.int32)
            for k in range(_TB // 16):
                v = plsc.load_gather(rowsv[b], [tidx[k], eidx])
                outv[b][e, pl.ds(k * 16, 16)] = v

    for i in range(_NIB):
        idx_copy(i, i).start()
    for j in range(_NBUF):
        idx_copy(j, j).wait()
        gather(j, j, j).start()

    def step(j, b, i, issue_gather, issue_idx, wait_out):
        gather(j, i, b).wait()
        if wait_out:
            for cp in out_copies(j - _NBUF, b):
                cp.wait()
        transpose(b)
        for cp in out_copies(j, b):
            cp.start()
        if issue_idx:
            idx_copy(j + _NIB, i).start()
        if issue_gather:
            jn = j + _NBUF
            idx_copy(jn, (i + _NBUF) % _NIB).wait()
            gather(jn, (i + _NBUF) % _NIB, b).start()

    for j in range(_NBUF):
        step(j, j, j, True, True, False)
    for j in range(_NBUF, _NIB):
        step(j, j % _NBUF, j, True, True, True)

    @pl.loop(_NIB, _MAIN, step=_NIB)
    def _(jj):
        for u in range(_NIB):
            step(jj + u, u % _NBUF, u, True, True, True)

    for j in range(_MAIN, _BPW - _NBUF):
        step(j, j % _NBUF, j % _NIB, True, False, True)
    for j in range(_BPW - _NBUF, _BPW):
        step(j, j % _NBUF, j % _NIB, False, False, True)

    for j in range(_BPW - _NBUF, _BPW):
        for cp in out_copies(j, j % _NBUF):
            cp.wait()


_sc_gather = functools.partial(
    pl.kernel,
    out_type=jax.ShapeDtypeStruct((_SEQ * 4 * _TB, 8, _TB), jnp.float32),
    mesh=plsc.VectorSubcoreMesh(core_axis_name="c", subcore_axis_name="s"),
    compiler_params=pltpu.CompilerParams(
        use_tc_tiling_on_sc=False, needs_layout_passes=False
    ),
    scratch_types=[
        [pltpu.VMEM((_TB,), jnp.int32) for _ in range(_NIB)],
        [pltpu.VMEM((_TB, _EMBED), jnp.float32) for _ in range(_NBUF)],
        [pltpu.VMEM((_EMBED, _TB), jnp.float32) for _ in range(_NBUF)],
        [pltpu.SemaphoreType.DMA for _ in range(_NIB)],
        [pltpu.SemaphoreType.DMA for _ in range(_NBUF)],
        [pltpu.SemaphoreType.DMA for _ in range(_NBUF)],
    ],
)(_sc_body)


@jax.jit
def kernel(tokenid, table):
    idx_t = jnp.swapaxes(tokenid, 0, 1)
    out2 = _sc_gather(table, idx_t)
    out5 = out2.reshape(_SEQ, 4, _TB, 8, _TB)
    return out5.transpose(2, 4, 0, 1, 3).reshape(_NTOK, _SEQ, _EMBED)


# speedup vs baseline: 1.8520x; 1.1266x over previous
import functools

import jax
import jax.numpy as jnp
from jax import lax
from jax.experimental import pallas as pl
from jax.experimental.pallas import tpu as pltpu
from jax.experimental.pallas import tpu_sc as plsc

_EMBED = 32
_SEQ = 50
_NTOK = 16384
_TB = 128
_NTB = _NTOK // _TB
_NBLK = _SEQ * _NTB
_NC = 2
_NS = 16
_NW = _NC * _NS
_BPW = _NBLK // _NW
_NBUF = 4
_NIB = 2 * _NBUF
_MAIN = _BPW - _NIB


def _sc_body(table_hbm, idx_hbm, out_hbm, idxv, rowsv, outv, isems, gsems, osems):
    wid = lax.axis_index("s") * _NC + lax.axis_index("c")
    base = wid * _BPW

    def blk(j):
        b = base + j
        return b >> 7, b & 127

    def idx_copy(j, i):
        s, t = blk(j)
        return pltpu.make_async_copy(
            idx_hbm.at[s, pl.ds(t * _TB, _TB)], idxv[i], isems[i]
        )

    def gather(j, i, b):
        return pltpu.make_async_copy(table_hbm.at[idxv[i]], rowsv[b], gsems[b])

    def out_copies(j, b):
        s, t = blk(j)
        return [
            pltpu.make_async_copy(
                outv[b].at[pl.ds(e8 * 8, 8)],
                out_hbm.at[s * (4 * _TB) + e8 * _TB + t],
                osems[b],
            )
            for e8 in range(4)
        ]

    iota_lo = lax.iota(jnp.int32, 16)
    iota_hi = iota_lo + 16

    def transpose(b):
        @pl.loop(0, _TB, unroll=8)
        def _(t):
            tfull = jnp.full((16,), t, jnp.int32)
            v0 = rowsv[b][t, pl.ds(0, 16)]
            v1 = rowsv[b][t, pl.ds(16, 16)]
            plsc.store_scatter(outv[b], [iota_lo, tfull], v0)
            plsc.store_scatter(outv[b], [iota_hi, tfull], v1)

    for i in range(_NIB):
        idx_copy(i, i).start()
    for j in range(_NBUF):
        idx_copy(j, j).wait()
        gather(j, j, j).start()

    def step(j, b, i, issue_gather, issue_idx, wait_out):
        gather(j, i, b).wait()
        if wait_out:
            for cp in out_copies(j - _NBUF, b):
                cp.wait()
        transpose(b)
        for cp in out_copies(j, b):
            cp.start()
        if issue_idx:
            idx_copy(j + _NIB, i).start()
        if issue_gather:
            jn = j + _NBUF
            idx_copy(jn, (i + _NBUF) % _NIB).wait()
            gather(jn, (i + _NBUF) % _NIB, b).start()

    for j in range(_NBUF):
        step(j, j, j, True, True, False)
    for j in range(_NBUF, _NIB):
        step(j, j % _NBUF, j, True, True, True)

    @pl.loop(_NIB, _MAIN, step=_NIB)
    def _(jj):
        for u in range(_NIB):
            step(jj + u, u % _NBUF, u, True, True, True)

    for j in range(_MAIN, _BPW - _NBUF):
        step(j, j % _NBUF, j % _NIB, True, False, True)
    for j in range(_BPW - _NBUF, _BPW):
        step(j, j % _NBUF, j % _NIB, False, False, True)

    for j in range(_BPW - _NBUF, _BPW):
        for cp in out_copies(j, j % _NBUF):
            cp.wait()


_sc_gather = functools.partial(
    pl.kernel,
    out_type=jax.ShapeDtypeStruct((_SEQ * 4 * _TB, 8, _TB), jnp.float32),
    mesh=plsc.VectorSubcoreMesh(core_axis_name="c", subcore_axis_name="s"),
    compiler_params=pltpu.CompilerParams(
        use_tc_tiling_on_sc=False, needs_layout_passes=False
    ),
    scratch_types=[
        [pltpu.VMEM((_TB,), jnp.int32) for _ in range(_NIB)],
        [pltpu.VMEM((_TB, _EMBED), jnp.float32) for _ in range(_NBUF)],
        [pltpu.VMEM((_EMBED, _TB), jnp.float32) for _ in range(_NBUF)],
        [pltpu.SemaphoreType.DMA for _ in range(_NIB)],
        [pltpu.SemaphoreType.DMA for _ in range(_NBUF)],
        [pltpu.SemaphoreType.DMA for _ in range(_NBUF)],
    ],
)(_sc_body)


@jax.jit
def kernel(tokenid, table):
    idx_t = jnp.swapaxes(tokenid, 0, 1)
    out2 = _sc_gather(table, idx_t)
    out5 = out2.reshape(_SEQ, 4, _TB, 8, _TB)
    return out5.transpose(2, 4, 0, 1, 3).reshape(_NTOK, _SEQ, _EMBED)


# speedup vs baseline: 1.8683x; 1.0088x over previous
import functools

import jax
import jax.numpy as jnp
from jax import lax
from jax.experimental import pallas as pl
from jax.experimental.pallas import tpu as pltpu
from jax.experimental.pallas import tpu_sc as plsc

_EMBED = 32
_SEQ = 50
_NTOK = 16384
_TB = 128
_NC = 2
_NS = 16
_NW = _NC * _NS
_WTOK = _NTOK // _NW
_NTI = _WTOK // _TB
_NBUF = 2


def _sc_body(table_hbm, idx_hbm, out_hbm, idxv, rowsv, outv, isem, gsems, osems):
    wid = lax.axis_index("s") * _NC + lax.axis_index("c")
    tw = wid * _NTI

    pltpu.sync_copy(idx_hbm.at[:, pl.ds(wid * _WTOK, _WTOK)], idxv)

    def gathers(s, b):
        return [
            pltpu.make_async_copy(
                table_hbm.at[idxv.at[s, pl.ds(ti * _TB, _TB)]],
                rowsv[b].at[pl.ds(ti * _TB, _TB)],
                gsems[b],
            )
            for ti in range(_NTI)
        ]

    def out_copy(s, b):
        return pltpu.make_async_copy(
            outv[b], out_hbm.at[s].at[:, pl.ds(tw, _NTI)], osems[b]
        )

    iota = lax.iota(jnp.int32, 16)
    e8 = [(iota + 16 * h) >> 3 for h in range(2)]
    e8i = [(iota + 16 * h) & 7 for h in range(2)]

    def transpose(b):
        @pl.loop(0, _WTOK, unroll=8)
        def _(t):
            tiv = jnp.full((16,), t >> 7, jnp.int32)
            tlv = jnp.full((16,), t & 127, jnp.int32)
            v0 = rowsv[b][t, pl.ds(0, 16)]
            v1 = rowsv[b][t, pl.ds(16, 16)]
            plsc.store_scatter(outv[b], [e8[0], tiv, e8i[0], tlv], v0)
            plsc.store_scatter(outv[b], [e8[1], tiv, e8i[1], tlv], v1)

    def step(s, b, issue_gather, wait_out):
        for cp in gathers(s, b):
            cp.wait()
        if wait_out:
            out_copy(s - _NBUF, b).wait()
        transpose(b)
        out_copy(s, b).start()
        if issue_gather:
            for cp in gathers(s + _NBUF, b):
                cp.start()

    for b in range(_NBUF):
        for cp in gathers(b, b):
            cp.start()

    for s in range(_NBUF):
        step(s, s, True, False)

    @pl.loop(_NBUF, _SEQ - _NBUF, step=_NBUF)
    def _(ss):
        for u in range(_NBUF):
            step(ss + u, u, True, True)

    for s in range(_SEQ - _NBUF, _SEQ):
        step(s, s % _NBUF, False, True)
    for s in range(_SEQ - _NBUF, _SEQ):
        out_copy(s, s % _NBUF).wait()


_sc_gather = functools.partial(
    pl.kernel,
    out_type=jax.ShapeDtypeStruct((_SEQ, 4, _TB, 8, _TB), jnp.float32),
    mesh=plsc.VectorSubcoreMesh(core_axis_name="c", subcore_axis_name="s"),
    compiler_params=pltpu.CompilerParams(
        use_tc_tiling_on_sc=False, needs_layout_passes=False
    ),
    scratch_types=[
        pltpu.VMEM((_SEQ, _WTOK), jnp.int32),
        [pltpu.VMEM((_WTOK, _EMBED), jnp.float32) for _ in range(_NBUF)],
        [pltpu.VMEM((4, _NTI, 8, _TB), jnp.float32) for _ in range(_NBUF)],
        pltpu.SemaphoreType.DMA,
        [pltpu.SemaphoreType.DMA for _ in range(_NBUF)],
        [pltpu.SemaphoreType.DMA for _ in range(_NBUF)],
    ],
)(_sc_body)


@jax.jit
def kernel(tokenid, table):
    idx_t = jnp.swapaxes(tokenid, 0, 1)
    out5 = _sc_gather(table, idx_t)
    return out5.transpose(2, 4, 0, 1, 3).reshape(_NTOK, _SEQ, _EMBED)


# speedup vs baseline: 2.1026x; 1.1254x over previous
import functools

import jax
import jax.numpy as jnp
from jax import lax
from jax.experimental import pallas as pl
from jax.experimental.pallas import tpu as pltpu
from jax.experimental.pallas import tpu_sc as plsc

_EMBED = 32
_SEQ = 50
_NTOK = 16384
_TB = 128
_NC = 2
_NS = 16
_NW = _NC * _NS
_WTOK = _NTOK // _NW
_NTI = _WTOK // _TB
_NBUF = 2


def _sc_body(table_hbm, idx_hbm, out_hbm, idxv, rowsv, outv, isem, gsems, osems):
    wid = lax.axis_index("s") * _NC + lax.axis_index("c")
    tw = wid * _NTI

    pltpu.sync_copy(idx_hbm.at[:, pl.ds(wid * _WTOK, _WTOK)], idxv)

    def gathers(s, b):
        return [
            pltpu.make_async_copy(
                table_hbm.at[idxv.at[s, pl.ds(ti * _TB, _TB)]],
                rowsv[b].at[pl.ds(ti * _TB, _TB)],
                gsems[b],
            )
            for ti in range(_NTI)
        ]

    def out_copy(s, b):
        return pltpu.make_async_copy(
            outv[b], out_hbm.at[s].at[:, pl.ds(tw, _NTI)], osems[b]
        )

    iota = lax.iota(jnp.int32, 16)
    e8 = [(iota + 16 * h) >> 3 for h in range(2)]
    e8i = [(iota + 16 * h) & 7 for h in range(2)]

    def transpose(b):
        @plsc.parallel_loop(0, _WTOK, 1, unroll=8)
        def _(t):
            tiv = jnp.full((16,), t >> 7, jnp.int32)
            tlv = jnp.full((16,), t & 127, jnp.int32)
            v0 = rowsv[b][t, pl.ds(0, 16)]
            v1 = rowsv[b][t, pl.ds(16, 16)]
            plsc.store_scatter(outv[b], [e8[0], tiv, e8i[0], tlv], v0)
            plsc.store_scatter(outv[b], [e8[1], tiv, e8i[1], tlv], v1)

    def step(s, b, issue_gather, wait_out):
        for cp in gathers(s, b):
            cp.wait()
        if wait_out:
            out_copy(s - _NBUF, b).wait()
        transpose(b)
        out_copy(s, b).start()
        if issue_gather:
            for cp in gathers(s + _NBUF, b):
                cp.start()

    for b in range(_NBUF):
        for cp in gathers(b, b):
            cp.start()

    for s in range(_NBUF):
        step(s, s, True, False)

    @pl.loop(_NBUF, _SEQ - _NBUF, step=_NBUF)
    def _(ss):
        for u in range(_NBUF):
            step(ss + u, u, True, True)

    for s in range(_SEQ - _NBUF, _SEQ):
        step(s, s % _NBUF, False, True)
    for s in range(_SEQ - _NBUF, _SEQ):
        out_copy(s, s % _NBUF).wait()


_sc_gather = functools.partial(
    pl.kernel,
    out_type=jax.ShapeDtypeStruct((_SEQ, 4, _TB, 8, _TB), jnp.float32),
    mesh=plsc.VectorSubcoreMesh(core_axis_name="c", subcore_axis_name="s"),
    compiler_params=pltpu.CompilerParams(
        use_tc_tiling_on_sc=False, needs_layout_passes=False
    ),
    scratch_types=[
        pltpu.VMEM((_SEQ, _WTOK), jnp.int32),
        [pltpu.VMEM((_WTOK, _EMBED), jnp.float32) for _ in range(_NBUF)],
        [pltpu.VMEM((4, _NTI, 8, _TB), jnp.float32) for _ in range(_NBUF)],
        pltpu.SemaphoreType.DMA,
        [pltpu.SemaphoreType.DMA for _ in range(_NBUF)],
        [pltpu.SemaphoreType.DMA for _ in range(_NBUF)],
    ],
)(_sc_body)


@jax.jit
def kernel(tokenid, table):
    idx_t = jnp.swapaxes(tokenid, 0, 1)
    out5 = _sc_gather(table, idx_t)
    return out5.transpose(2, 4, 0, 1, 3).reshape(_NTOK, _SEQ, _EMBED)
